# baseline (speedup 1.0000x reference)
import functools

import jax
import jax.numpy as jnp
from jax import lax
from jax.experimental import pallas as pl
from jax.experimental.pallas import tpu as pltpu
from jax.experimental.pallas import tpu_sc as plsc

_PZBL = 0.23
_A0 = 0.4685
_C = (0.02817, 0.28022, 0.50986, 0.18175)
_D = (-0.20162, -0.4029, -0.94229, -3.1998)

_SPAD = 128


def _species_prep(z_ref, qq_ref, zq_ref, zp_ref):
    z = z_ref[...]
    zq_ref[...] = z * jnp.sqrt(qq_ref[...])
    zp_ref[...] = jnp.exp(jnp.log(z) * jnp.float32(_PZBL)) * jnp.float32(1.0 / _A0)


def _make_sc_kernel(n_nodes, epw):
    info = plsc.get_sparse_core_info()
    nc, ns, L = info.num_cores, info.num_subcores, info.num_lanes

    mesh = plsc.VectorSubcoreMesh(core_axis_name="c", subcore_axis_name="s")

    @functools.partial(
        pl.kernel,
        mesh=mesh,
        compiler_params=pltpu.CompilerParams(needs_layout_passes=False),
        out_type=jax.ShapeDtypeStruct((nc * ns * epw,), jnp.float32),
        scratch_types=[
            pltpu.VMEM((n_nodes,), jnp.int32),
            pltpu.VMEM((_SPAD,), jnp.float32),
            pltpu.VMEM((_SPAD,), jnp.float32),
            pltpu.VMEM((epw,), jnp.int32),
            pltpu.VMEM((epw,), jnp.int32),
            pltpu.VMEM((epw,), jnp.float32),
            pltpu.VMEM((epw,), jnp.float32),
        ],
    )
    def zbl_sc(types_hbm, zq_hbm, zp_hbm, ei_hbm, ej_hbm, r_hbm, out_hbm,
               types_v, zq_v, zp_v, ei_v, ej_v, r_v, out_v):
        wid = lax.axis_index("s") * nc + lax.axis_index("c")
        base = wid * epw
        pltpu.sync_copy(types_hbm, types_v)
        pltpu.sync_copy(zq_hbm, zq_v)
        pltpu.sync_copy(zp_hbm, zp_v)
        pltpu.sync_copy(ei_hbm.at[pl.ds(base, epw)], ei_v)
        pltpu.sync_copy(ej_hbm.at[pl.ds(base, epw)], ej_v)
        pltpu.sync_copy(r_hbm.at[pl.ds(base, epw)], r_v)

        c1, c2, c3, c4 = (jnp.float32(c) for c in _C)
        d1, d2, d3, d4 = (jnp.float32(d) for d in _D)

        @plsc.parallel_loop(0, epw, step=L, unroll=8)
        def body(off):
            iv = ei_v[pl.ds(off, L)]
            jv = ej_v[pl.ds(off, L)]
            rv = r_v[pl.ds(off, L)]
            ti = plsc.load_gather(types_v, [iv])
            tj = plsc.load_gather(types_v, [jv])
            zi = plsc.load_gather(zq_v, [ti])
            zj = plsc.load_gather(zq_v, [tj])
            pi = plsc.load_gather(zp_v, [ti])
            pj = plsc.load_gather(zp_v, [tj])
            x = (pi + pj) * rv
            psi = (c1 * jnp.exp(d1 * x) + c2 * jnp.exp(d2 * x)
                   + c3 * jnp.exp(d3 * x) + c4 * jnp.exp(d4 * x))
            out_v[pl.ds(off, L)] = (zi * zj / rv) * psi
        pltpu.sync_copy(out_v, out_hbm.at[pl.ds(base, epw)])

    return zbl_sc


def kernel(Z, r, atom_types, edge_index, qqr2exesquare):
    n_edges = r.shape[0]
    n_species = Z.shape[0]
    n_nodes = atom_types.shape[0]
    assert n_edges % (32 * 16) == 0

    types32 = atom_types.astype(jnp.int32)
    ei = edge_index[0].astype(jnp.int32)
    ej = edge_index[1].astype(jnp.int32)

    z_pad = jnp.pad(Z.astype(jnp.float32), (0, _SPAD - n_species),
                    constant_values=1.0).reshape(1, _SPAD)
    qq_b = jnp.broadcast_to(jnp.float32(qqr2exesquare), (1, _SPAD))

    zq, zp = pl.pallas_call(
        _species_prep,
        out_shape=[
            jax.ShapeDtypeStruct((1, _SPAD), jnp.float32),
            jax.ShapeDtypeStruct((1, _SPAD), jnp.float32),
        ],
    )(z_pad, qq_b)
    zq = zq.reshape(_SPAD)
    zp = zp.reshape(_SPAD)

    epw = n_edges // 32
    eng = _make_sc_kernel(n_nodes, epw)(types32, zq, zp, ei, ej, r)
    return eng


# speedup vs baseline: 1.0536x; 1.0536x over previous
import functools

import jax
import jax.numpy as jnp
from jax import lax
from jax.experimental import pallas as pl
from jax.experimental.pallas import tpu as pltpu
from jax.experimental.pallas import tpu_sc as plsc

_PZBL = 0.23
_A0 = 0.4685
_C = (0.02817, 0.28022, 0.50986, 0.18175)
_D = (-0.20162, -0.4029, -0.94229, -3.1998)

_SPAD = 128


def _species_prep(z_ref, qq_ref, zq_ref, zp_ref):
    z = z_ref[...]
    zq_ref[...] = z * jnp.sqrt(qq_ref[...])
    zp_ref[...] = jnp.exp(jnp.log(z) * jnp.float32(_PZBL)) * jnp.float32(1.0 / _A0)


def _make_sc_kernel(n_nodes, epw):
    info = plsc.get_sparse_core_info()
    nc, ns, L = info.num_cores, info.num_subcores, info.num_lanes

    mesh = plsc.VectorSubcoreMesh(core_axis_name="c", subcore_axis_name="s")

    @functools.partial(
        pl.kernel,
        mesh=mesh,
        compiler_params=pltpu.CompilerParams(needs_layout_passes=False),
        out_type=jax.ShapeDtypeStruct((nc * ns * epw,), jnp.float32),
        scratch_types=[
            pltpu.VMEM((n_nodes,), jnp.int32),
            pltpu.VMEM((_SPAD,), jnp.float32),
            pltpu.VMEM((_SPAD,), jnp.float32),
            pltpu.VMEM((n_nodes,), jnp.float32),
            pltpu.VMEM((n_nodes,), jnp.float32),
            pltpu.VMEM((epw,), jnp.int32),
            pltpu.VMEM((epw,), jnp.int32),
            pltpu.VMEM((epw,), jnp.float32),
            pltpu.VMEM((epw,), jnp.float32),
            pltpu.SemaphoreType.DMA,
        ],
    )
    def zbl_sc(types_hbm, zq_hbm, zp_hbm, ei_hbm, ej_hbm, r_hbm, out_hbm,
               types_v, zq_v, zp_v, za_v, zpn_v, ei_v, ej_v, r_v, out_v, sem):
        wid = lax.axis_index("s") * nc + lax.axis_index("c")
        base = wid * epw
        cp1 = pltpu.async_copy(ei_hbm.at[pl.ds(base, epw)], ei_v, sem)
        cp2 = pltpu.async_copy(ej_hbm.at[pl.ds(base, epw)], ej_v, sem)
        cp3 = pltpu.async_copy(r_hbm.at[pl.ds(base, epw)], r_v, sem)
        pltpu.sync_copy(types_hbm, types_v)
        pltpu.sync_copy(zq_hbm, zq_v)
        pltpu.sync_copy(zp_hbm, zp_v)

        @plsc.parallel_loop(0, n_nodes, step=L, unroll=4)
        def prologue(off):
            tv = types_v[pl.ds(off, L)]
            za_v[pl.ds(off, L)] = plsc.load_gather(zq_v, [tv])
            zpn_v[pl.ds(off, L)] = plsc.load_gather(zp_v, [tv])

        cp1.wait()
        cp2.wait()
        cp3.wait()

        c1, c2, c3, c4 = (jnp.float32(c) for c in _C)
        d1, d2, d3, d4 = (jnp.float32(d) for d in _D)

        @plsc.parallel_loop(0, epw, step=L, unroll=4)
        def body(off):
            iv = ei_v[pl.ds(off, L)]
            jv = ej_v[pl.ds(off, L)]
            rv = r_v[pl.ds(off, L)]
            zi = plsc.load_gather(za_v, [iv])
            zj = plsc.load_gather(za_v, [jv])
            pi = plsc.load_gather(zpn_v, [iv])
            pj = plsc.load_gather(zpn_v, [jv])
            x = (pi + pj) * rv
            psi = (c1 * jnp.exp(d1 * x) + c2 * jnp.exp(d2 * x)
                   + c3 * jnp.exp(d3 * x) + c4 * jnp.exp(d4 * x))
            out_v[pl.ds(off, L)] = (zi * zj / rv) * psi
        pltpu.sync_copy(out_v, out_hbm.at[pl.ds(base, epw)])

    return zbl_sc


def kernel(Z, r, atom_types, edge_index, qqr2exesquare):
    n_edges = r.shape[0]
    n_species = Z.shape[0]
    n_nodes = atom_types.shape[0]
    assert n_edges % (32 * 16) == 0

    types32 = atom_types.astype(jnp.int32)
    ei = edge_index[0].astype(jnp.int32)
    ej = edge_index[1].astype(jnp.int32)

    z_pad = jnp.pad(Z.astype(jnp.float32), (0, _SPAD - n_species),
                    constant_values=1.0).reshape(1, _SPAD)
    qq_b = jnp.broadcast_to(jnp.float32(qqr2exesquare), (1, _SPAD))

    zq, zp = pl.pallas_call(
        _species_prep,
        out_shape=[
            jax.ShapeDtypeStruct((1, _SPAD), jnp.float32),
            jax.ShapeDtypeStruct((1, _SPAD), jnp.float32),
        ],
    )(z_pad, qq_b)
    zq = zq.reshape(_SPAD)
    zp = zp.reshape(_SPAD)

    epw = n_edges // 32
    eng = _make_sc_kernel(n_nodes, epw)(types32, zq, zp, ei, ej, r)
    return eng


# speedup vs baseline: 1.2419x; 1.1788x over previous
import functools
import math

import jax
import jax.numpy as jnp
from jax import lax
from jax.experimental import pallas as pl
from jax.experimental.pallas import tpu as pltpu
from jax.experimental.pallas import tpu_sc as plsc

_PZBL = 0.23
_A0 = 0.4685
_C = (0.02817, 0.28022, 0.50986, 0.18175)
_D = (-0.20162, -0.4029, -0.94229, -3.1998)

_SPAD = 128
_LN2 = math.log(2.0)


def _make_sc_kernel(n_nodes, n_edges):
    info = plsc.get_sparse_core_info()
    nc, ns, L = info.num_cores, info.num_subcores, info.num_lanes
    epw = n_edges // (nc * ns)
    mesh = plsc.VectorSubcoreMesh(core_axis_name="c", subcore_axis_name="s")

    @functools.partial(
        pl.kernel,
        mesh=mesh,
        compiler_params=pltpu.CompilerParams(needs_layout_passes=False),
        out_type=jax.ShapeDtypeStruct((n_edges,), jnp.float32),
        scratch_types=[
            pltpu.VMEM((n_nodes,), jnp.int32),
            pltpu.VMEM((_SPAD,), jnp.float32),
            pltpu.VMEM((_SPAD,), jnp.float32),
            pltpu.VMEM((L,), jnp.float32),
            pltpu.VMEM((n_nodes,), jnp.float32),
            pltpu.VMEM((n_nodes,), jnp.float32),
            pltpu.VMEM((epw,), jnp.int32),
            pltpu.VMEM((epw,), jnp.int32),
            pltpu.VMEM((epw,), jnp.float32),
            pltpu.VMEM((epw,), jnp.float32),
            pltpu.SemaphoreType.DMA,
        ],
    )
    def zbl_sc(types_hbm, z_hbm, qq_hbm, e_hbm, r_hbm, out_hbm,
               types_v, z_v, zp_v, qq_v, za_v, zpn_v, ei_v, ej_v, r_v, out_v,
               sem):
        wid = lax.axis_index("s") * nc + lax.axis_index("c")
        base = wid * epw
        cp1 = pltpu.async_copy(e_hbm.at[pl.ds(base, epw)], ei_v, sem)
        cp2 = pltpu.async_copy(e_hbm.at[pl.ds(n_edges + base, epw)], ej_v, sem)
        cp3 = pltpu.async_copy(r_hbm.at[pl.ds(base, epw)], r_v, sem)
        pltpu.sync_copy(types_hbm, types_v)
        pltpu.sync_copy(z_hbm, z_v)
        pltpu.sync_copy(qq_hbm, qq_v)

        inv_a0 = jnp.float32(1.0 / _A0)
        ln2 = jnp.float32(_LN2)
        p = jnp.float32(_PZBL)
        one = jnp.float32(1.0)

        @plsc.parallel_loop(0, _SPAD, step=L, unroll=2)
        def species(off):
            z = z_v[pl.ds(off, L)]
            bits = plsc.bitcast(z, jnp.int32)
            e = (lax.shift_right_arithmetic(bits, 23) - 127).astype(jnp.float32)
            y = e * ln2
            for _ in range(4):
                y = y + (z * jnp.exp(-y) - one)
            zp_v[pl.ds(off, L)] = jnp.exp(p * y) * inv_a0

        @plsc.parallel_loop(0, n_nodes, step=L, unroll=4)
        def nodes(off):
            tv = types_v[pl.ds(off, L)]
            za_v[pl.ds(off, L)] = plsc.load_gather(z_v, [tv])
            zpn_v[pl.ds(off, L)] = plsc.load_gather(zp_v, [tv])

        cp1.wait()
        cp2.wait()
        cp3.wait()

        qv = qq_v[pl.ds(0, L)]
        cq1, cq2, cq3, cq4 = (jnp.float32(c) * qv for c in _C)
        d1, d2, d3, d4 = (jnp.float32(d) for d in _D)

        @plsc.parallel_loop(0, epw, step=L, unroll=4)
        def body(off):
            iv = ei_v[pl.ds(off, L)]
            jv = ej_v[pl.ds(off, L)]
            rv = r_v[pl.ds(off, L)]
            zi = plsc.load_gather(za_v, [iv])
            zj = plsc.load_gather(za_v, [jv])
            pi = plsc.load_gather(zpn_v, [iv])
            pj = plsc.load_gather(zpn_v, [jv])
            x = (pi + pj) * rv
            psi = (cq1 * jnp.exp(d1 * x) + cq2 * jnp.exp(d2 * x)
                   + cq3 * jnp.exp(d3 * x) + cq4 * jnp.exp(d4 * x))
            out_v[pl.ds(off, L)] = (zi * zj / rv) * psi

        pltpu.sync_copy(out_v, out_hbm.at[pl.ds(base, epw)])

    return zbl_sc


def kernel(Z, r, atom_types, edge_index, qqr2exesquare):
    n_edges = r.shape[0]
    n_species = Z.shape[0]
    n_nodes = atom_types.shape[0]
    assert n_edges % (32 * 16) == 0 and n_nodes % 16 == 0

    types32 = atom_types.astype(jnp.int32)
    eflat = edge_index.astype(jnp.int32).reshape(-1)
    z_pad = jnp.pad(Z.astype(jnp.float32), (0, _SPAD - n_species),
                    constant_values=1.0)
    qq_b = jnp.broadcast_to(jnp.float32(qqr2exesquare), (16,))

    return _make_sc_kernel(n_nodes, n_edges)(types32, z_pad, qq_b, eflat, r)


# speedup vs baseline: 1.2432x; 1.0010x over previous
import functools
import math

import jax
import jax.numpy as jnp
from jax import lax
from jax.experimental import pallas as pl
from jax.experimental.pallas import tpu as pltpu
from jax.experimental.pallas import tpu_sc as plsc

_PZBL = 0.23
_A0 = 0.4685
_C = (0.02817, 0.28022, 0.50986, 0.18175)
_D = (-0.20162, -0.4029, -0.94229, -3.1998)

_SPAD = 128
_LN2 = math.log(2.0)


def _make_sc_kernel(n_nodes, n_edges):
    info = plsc.get_sparse_core_info()
    nc, ns, L = info.num_cores, info.num_subcores, info.num_lanes
    epw = n_edges // (nc * ns)
    mesh = plsc.VectorSubcoreMesh(core_axis_name="c", subcore_axis_name="s")

    @functools.partial(
        pl.kernel,
        mesh=mesh,
        compiler_params=pltpu.CompilerParams(needs_layout_passes=False),
        out_type=jax.ShapeDtypeStruct((n_edges,), jnp.float32),
        scratch_types=[
            pltpu.VMEM((n_nodes,), jnp.int32),
            pltpu.VMEM((_SPAD,), jnp.float32),
            pltpu.VMEM((_SPAD,), jnp.float32),
            pltpu.VMEM((L,), jnp.float32),
            pltpu.VMEM((n_nodes,), jnp.float32),
            pltpu.VMEM((n_nodes,), jnp.float32),
            pltpu.VMEM((epw,), jnp.int32),
            pltpu.VMEM((epw,), jnp.int32),
            pltpu.VMEM((epw,), jnp.float32),
            pltpu.VMEM((epw,), jnp.float32),
            pltpu.SemaphoreType.DMA,
        ],
    )
    def zbl_sc(types_hbm, z_hbm, qq_hbm, e_hbm, r_hbm, out_hbm,
               types_v, z_v, zp_v, qq_v, za_v, zpn_v, ei_v, ej_v, r_v, out_v,
               sem):
        wid = lax.axis_index("s") * nc + lax.axis_index("c")
        base = wid * epw
        half = epw // 2
        cps = []
        for h in range(2):
            hb = base + h * half
            cps.append((
                pltpu.async_copy(e_hbm.at[pl.ds(hb, half)],
                                 ei_v.at[pl.ds(h * half, half)], sem),
                pltpu.async_copy(e_hbm.at[pl.ds(n_edges + hb, half)],
                                 ej_v.at[pl.ds(h * half, half)], sem),
                pltpu.async_copy(r_hbm.at[pl.ds(hb, half)],
                                 r_v.at[pl.ds(h * half, half)], sem),
            ))
        pltpu.sync_copy(types_hbm, types_v)
        pltpu.sync_copy(z_hbm, z_v)
        pltpu.sync_copy(qq_hbm, qq_v)

        inv_a0 = jnp.float32(1.0 / _A0)
        ln2 = jnp.float32(_LN2)
        p = jnp.float32(_PZBL)
        one = jnp.float32(1.0)

        @plsc.parallel_loop(0, _SPAD, step=L, unroll=2)
        def species(off):
            z = z_v[pl.ds(off, L)]
            bits = plsc.bitcast(z, jnp.int32)
            e = (lax.shift_right_arithmetic(bits, 23) - 127).astype(jnp.float32)
            y = e * ln2
            for _ in range(4):
                y = y + (z * jnp.exp(-y) - one)
            zp_v[pl.ds(off, L)] = jnp.exp(p * y) * inv_a0

        @plsc.parallel_loop(0, n_nodes, step=L, unroll=4)
        def nodes(off):
            tv = types_v[pl.ds(off, L)]
            za_v[pl.ds(off, L)] = plsc.load_gather(z_v, [tv])
            zpn_v[pl.ds(off, L)] = plsc.load_gather(zp_v, [tv])

        qv = qq_v[pl.ds(0, L)]
        cq1, cq2, cq3, cq4 = (jnp.float32(c) * qv for c in _C)
        d1, d2, d3, d4 = (jnp.float32(d) for d in _D)

        wb = []
        for h in range(2):
            for cp in cps[h]:
                cp.wait()
            hoff = h * half

            @plsc.parallel_loop(hoff, hoff + half, step=L, unroll=4)
            def body(off):
                iv = ei_v[pl.ds(off, L)]
                jv = ej_v[pl.ds(off, L)]
                rv = r_v[pl.ds(off, L)]
                zi = plsc.load_gather(za_v, [iv])
                zj = plsc.load_gather(za_v, [jv])
                pi = plsc.load_gather(zpn_v, [iv])
                pj = plsc.load_gather(zpn_v, [jv])
                x = (pi + pj) * rv
                psi = (cq1 * jnp.exp(d1 * x) + cq2 * jnp.exp(d2 * x)
                       + cq3 * jnp.exp(d3 * x) + cq4 * jnp.exp(d4 * x))
                out_v[pl.ds(off, L)] = (zi * zj / rv) * psi

            wb.append(pltpu.async_copy(
                out_v.at[pl.ds(hoff, half)],
                out_hbm.at[pl.ds(base + hoff, half)], sem))
        for w in wb:
            w.wait()

    return zbl_sc


def kernel(Z, r, atom_types, edge_index, qqr2exesquare):
    n_edges = r.shape[0]
    n_species = Z.shape[0]
    n_nodes = atom_types.shape[0]
    assert n_edges % (32 * 16) == 0 and n_nodes % 16 == 0

    types32 = atom_types.astype(jnp.int32)
    eflat = edge_index.astype(jnp.int32).reshape(-1)
    z_pad = jnp.pad(Z.astype(jnp.float32), (0, _SPAD - n_species),
                    constant_values=1.0)
    qq_b = jnp.broadcast_to(jnp.float32(qqr2exesquare), (16,))

    return _make_sc_kernel(n_nodes, n_edges)(types32, z_pad, qq_b, eflat, r)


# speedup vs baseline: 1.2485x; 1.0042x over previous
import functools
import math

import jax
import jax.numpy as jnp
from jax import lax
from jax.experimental import pallas as pl
from jax.experimental.pallas import tpu as pltpu
from jax.experimental.pallas import tpu_sc as plsc

_PZBL = 0.23
_A0 = 0.4685
_C = (0.02817, 0.28022, 0.50986, 0.18175)
_D = (-0.20162, -0.4029, -0.94229, -3.1998)

_SPAD = 128
_LN2 = math.log(2.0)


def _make_sc_kernel(n_nodes, n_edges):
    info = plsc.get_sparse_core_info()
    nc, ns, L = info.num_cores, info.num_subcores, info.num_lanes
    epw = n_edges // (nc * ns)
    mesh = plsc.VectorSubcoreMesh(core_axis_name="c", subcore_axis_name="s")

    @functools.partial(
        pl.kernel,
        mesh=mesh,
        compiler_params=pltpu.CompilerParams(needs_layout_passes=False),
        out_type=jax.ShapeDtypeStruct((n_edges,), jnp.float32),
        scratch_types=[
            pltpu.VMEM((n_nodes,), jnp.int32),
            pltpu.VMEM((_SPAD,), jnp.float32),
            pltpu.VMEM((_SPAD,), jnp.float32),
            pltpu.VMEM((L,), jnp.float32),
            pltpu.VMEM((n_nodes,), jnp.float32),
            pltpu.VMEM((n_nodes,), jnp.float32),
            pltpu.VMEM((epw,), jnp.int32),
            pltpu.VMEM((epw,), jnp.int32),
            pltpu.VMEM((epw,), jnp.float32),
            pltpu.VMEM((epw,), jnp.float32),
            pltpu.VMEM_SHARED((n_nodes,), jnp.float32),
            pltpu.VMEM_SHARED((n_nodes,), jnp.float32),
            pltpu.SemaphoreType.DMA,
        ],
    )
    def zbl_sc(types_hbm, z_hbm, qq_hbm, e_hbm, r_hbm, out_hbm,
               types_v, z_v, zp_v, qq_v, za_v, zpn_v, ei_v, ej_v, r_v, out_v,
               za_sh, zp_sh, sem):
        wid = lax.axis_index("s") * nc + lax.axis_index("c")
        base = wid * epw
        half = epw // 2
        cps = []
        for h in range(2):
            hb = base + h * half
            cps.append((
                pltpu.async_copy(e_hbm.at[pl.ds(hb, half)],
                                 ei_v.at[pl.ds(h * half, half)], sem),
                pltpu.async_copy(e_hbm.at[pl.ds(n_edges + hb, half)],
                                 ej_v.at[pl.ds(h * half, half)], sem),
                pltpu.async_copy(r_hbm.at[pl.ds(hb, half)],
                                 r_v.at[pl.ds(h * half, half)], sem),
            ))
        pltpu.sync_copy(types_hbm, types_v)
        pltpu.sync_copy(z_hbm, z_v)
        pltpu.sync_copy(qq_hbm, qq_v)

        inv_a0 = jnp.float32(1.0 / _A0)
        ln2 = jnp.float32(_LN2)
        p = jnp.float32(_PZBL)
        one = jnp.float32(1.0)

        @plsc.parallel_loop(0, _SPAD, step=L, unroll=2)
        def species(off):
            z = z_v[pl.ds(off, L)]
            bits = plsc.bitcast(z, jnp.int32)
            e = (lax.shift_right_arithmetic(bits, 23) - 127).astype(jnp.float32)
            y = e * ln2
            for _ in range(4):
                y = y + (z * jnp.exp(-y) - one)
            zp_v[pl.ds(off, L)] = jnp.exp(p * y) * inv_a0

        per_tile = -(-(n_nodes // L) // ns) * L
        sid = lax.axis_index("s")
        start = jnp.minimum(sid * per_tile, n_nodes - per_tile)

        @plsc.parallel_loop(0, per_tile, step=L, unroll=4)
        def nodes(off):
            tv = types_v[pl.ds(start + off, L)]
            za_v[pl.ds(start + off, L)] = plsc.load_gather(z_v, [tv])
            zpn_v[pl.ds(start + off, L)] = plsc.load_gather(zp_v, [tv])

        pltpu.sync_copy(za_v.at[pl.ds(start, per_tile)],
                        za_sh.at[pl.ds(start, per_tile)])
        pltpu.sync_copy(zpn_v.at[pl.ds(start, per_tile)],
                        zp_sh.at[pl.ds(start, per_tile)])
        plsc.subcore_barrier()
        pltpu.sync_copy(za_sh, za_v)
        pltpu.sync_copy(zp_sh, zpn_v)

        qv = qq_v[pl.ds(0, L)]
        cq1, cq2, cq3, cq4 = (jnp.float32(c) * qv for c in _C)
        d1, d2, d3, d4 = (jnp.float32(d) for d in _D)

        wb = []
        for h in range(2):
            for cp in cps[h]:
                cp.wait()
            hoff = h * half

            @plsc.parallel_loop(hoff, hoff + half, step=L, unroll=4)
            def body(off):
                iv = ei_v[pl.ds(off, L)]
                jv = ej_v[pl.ds(off, L)]
                rv = r_v[pl.ds(off, L)]
                zi = plsc.load_gather(za_v, [iv])
                zj = plsc.load_gather(za_v, [jv])
                pi = plsc.load_gather(zpn_v, [iv])
                pj = plsc.load_gather(zpn_v, [jv])
                x = (pi + pj) * rv
                psi = (cq1 * jnp.exp(d1 * x) + cq2 * jnp.exp(d2 * x)
                       + cq3 * jnp.exp(d3 * x) + cq4 * jnp.exp(d4 * x))
                out_v[pl.ds(off, L)] = (zi * zj / rv) * psi

            wb.append(pltpu.async_copy(
                out_v.at[pl.ds(hoff, half)],
                out_hbm.at[pl.ds(base + hoff, half)], sem))
        for w in wb:
            w.wait()

    return zbl_sc


def kernel(Z, r, atom_types, edge_index, qqr2exesquare):
    n_edges = r.shape[0]
    n_species = Z.shape[0]
    n_nodes = atom_types.shape[0]
    assert n_edges % (32 * 16) == 0 and n_nodes % 16 == 0

    types32 = atom_types.astype(jnp.int32)
    eflat = edge_index.astype(jnp.int32).reshape(-1)
    z_pad = jnp.pad(Z.astype(jnp.float32), (0, _SPAD - n_species),
                    constant_values=1.0)
    qq_b = jnp.broadcast_to(jnp.float32(qqr2exesquare), (16,))

    return _make_sc_kernel(n_nodes, n_edges)(types32, z_pad, qq_b, eflat, r)
